# baseline (speedup 1.0000x reference)
import functools

import jax
import jax.numpy as jnp
from jax import lax
from jax.experimental import pallas as pl
from jax.experimental.pallas import tpu as pltpu
from jax.experimental.pallas import tpu_sc as plsc

_NC = 2
_NS = 16
_NW = _NC * _NS
_NBUF = 4
_L = 16


@functools.lru_cache(maxsize=None)
def _make_gather(B: int, T: int, D: int, V: int):
    assert B == 128 * _NW and D % 16 == 0
    DP = 128
    R = D // 8
    mesh = plsc.VectorSubcoreMesh(core_axis_name="c", subcore_axis_name="s")

    @functools.partial(
        pl.kernel,
        out_type=jax.ShapeDtypeStruct((T, R, _NW, 8, 128), jnp.float32),
        mesh=mesh,
        scratch_types=[
            pltpu.VMEM((T, 128), jnp.int32),
            pltpu.VMEM((_NBUF, 128, DP), jnp.int32),
            pltpu.VMEM((2, R, 8, 128), jnp.float32),
            pltpu.SemaphoreType.DMA((_NBUF,)),
            pltpu.SemaphoreType.DMA((2,)),
        ],
        compiler_params=pltpu.CompilerParams(
            use_tc_tiling_on_sc=False, needs_layout_passes=False
        ),
    )
    def k(table_hbm, idx_hbm, out_hbm, idx_v, rows_v, tr_v, gsem, tsem):
        wid = lax.axis_index("s") * _NC + lax.axis_index("c")
        pltpu.sync_copy(idx_hbm.at[:, pl.ds(wid * 128, 128)], idx_v)

        def start_gather(g, buf):
            pltpu.async_copy(
                table_hbm.at[idx_v.at[g]], rows_v.at[buf], gsem.at[buf]
            )

        def store_descr(g, buf):
            return pltpu.make_async_copy(
                tr_v.at[buf], out_hbm.at[g, :, wid], tsem.at[buf]
            )

        for p in range(_NBUF - 1):
            start_gather(p, p)

        @pl.loop(0, T)
        def _(g):
            gbuf = lax.rem(g, _NBUF)
            tbuf = lax.rem(g, 2)

            @pl.when(g + _NBUF - 1 < T)
            def _():
                start_gather(g + _NBUF - 1, lax.rem(g + _NBUF - 1, _NBUF))

            pltpu.make_async_copy(
                table_hbm.at[idx_v.at[g]], rows_v.at[gbuf], gsem.at[gbuf]
            ).wait()

            @pl.when(g >= 2)
            def _():
                store_descr(g - 2, tbuf).wait()

            rows = rows_v.at[gbuf]
            tr3 = tr_v.at[tbuf]

            @pl.loop(0, _L)
            def _(j):
                iota = lax.iota(jnp.int32, _L)
                perm = jnp.bitwise_and(iota + j, _L - 1)
                for p0 in range(0, D // 2, _L):
                    pcol = p0 + perm
                    d_lo = 2 * pcol
                    r_lo = lax.shift_right_logical(d_lo, 3)
                    s_lo = jnp.bitwise_and(d_lo, 7)
                    r_hi = lax.shift_right_logical(d_lo + 1, 3)
                    s_hi = jnp.bitwise_and(d_lo + 1, 7)
                    for l0 in range(0, 128, _L):
                        lanes = l0 + lax.iota(jnp.int32, _L)
                        w32 = plsc.load_gather(rows, [lanes, pcol])
                        lo = lax.shift_left(w32, 16)
                        hi = jnp.bitwise_and(w32, jnp.int32(-65536))
                        plsc.store_scatter(
                            tr3,
                            [r_lo, s_lo, lanes],
                            lax.bitcast_convert_type(lo, jnp.float32),
                        )
                        plsc.store_scatter(
                            tr3,
                            [r_hi, s_hi, lanes],
                            lax.bitcast_convert_type(hi, jnp.float32),
                        )

            store_descr(g, tbuf).start()

        store_descr(T - 2, lax.rem(T - 2, 2)).wait()
        store_descr(T - 1, lax.rem(T - 1, 2)).wait()

    return k


def kernel(x, W):
    B, T = x.shape
    V, D = W.shape
    Wb = lax.bitcast_convert_type(W, jnp.uint32)
    pairs = (Wb[:, 0::2] >> 16) | (Wb[:, 1::2] & jnp.uint32(0xFFFF0000))
    Wi = lax.bitcast_convert_type(
        jnp.pad(pairs, ((0, 0), (0, 128 - D // 2))), jnp.int32
    )
    xT = x.T.astype(jnp.int32)
    out5 = _make_gather(B, T, D, V)(Wi, xT)
    return out5.transpose(2, 4, 0, 1, 3).reshape(B, T, D)


# speedup vs baseline: 11.4744x; 11.4744x over previous
import functools

import jax
import jax.numpy as jnp
from jax import lax
from jax.experimental import pallas as pl
from jax.experimental.pallas import tpu as pltpu
from jax.experimental.pallas import tpu_sc as plsc

_NC = 2
_NS = 16
_NW = _NC * _NS
_NBUF = 4
_L = 16


@functools.lru_cache(maxsize=None)
def _make_gather(B: int, T: int, D: int, V: int):
    assert B == 128 * _NW and D % 16 == 0
    DP = D
    R = D // 8
    mesh = plsc.VectorSubcoreMesh(core_axis_name="c", subcore_axis_name="s")

    @functools.partial(
        pl.kernel,
        out_type=jax.ShapeDtypeStruct((T, R, _NW, 8, 128), jnp.float32),
        mesh=mesh,
        scratch_types=[
            pltpu.VMEM((T, 128), jnp.int32),
            pltpu.VMEM((_NBUF, 128, DP), jnp.int32),
            pltpu.VMEM((2, R, 8, 128), jnp.float32),
            pltpu.SemaphoreType.DMA((_NBUF,)),
            pltpu.SemaphoreType.DMA((2,)),
        ],
        compiler_params=pltpu.CompilerParams(
            use_tc_tiling_on_sc=False, needs_layout_passes=False
        ),
    )
    def k(table_hbm, idx_hbm, out_hbm, idx_v, rows_v, tr_v, gsem, tsem):
        wid = lax.axis_index("s") * _NC + lax.axis_index("c")
        pltpu.sync_copy(idx_hbm.at[:, pl.ds(wid * 128, 128)], idx_v)

        def start_gather(g, buf):
            pltpu.async_copy(
                table_hbm.at[idx_v.at[g]], rows_v.at[buf], gsem.at[buf]
            )

        def store_descr(g, buf):
            return pltpu.make_async_copy(
                tr_v.at[buf], out_hbm.at[g, :, wid], tsem.at[buf]
            )

        for p in range(_NBUF - 1):
            start_gather(p, p)

        @pl.loop(0, T)
        def _(g):
            gbuf = lax.rem(g, _NBUF)
            tbuf = lax.rem(g, 2)

            @pl.when(g + _NBUF - 1 < T)
            def _():
                start_gather(g + _NBUF - 1, lax.rem(g + _NBUF - 1, _NBUF))

            pltpu.make_async_copy(
                table_hbm.at[idx_v.at[g]], rows_v.at[gbuf], gsem.at[gbuf]
            ).wait()

            @pl.when(g >= 2)
            def _():
                store_descr(g - 2, tbuf).wait()

            rows = rows_v.at[gbuf]
            tr3 = tr_v.at[tbuf]

            @pl.loop(0, _L)
            def _(j):
                iota = lax.iota(jnp.int32, _L)
                perm = jnp.bitwise_and(iota + j, _L - 1)
                for d0 in range(0, D, _L):
                    cols = d0 + perm
                    r_idx = lax.shift_right_logical(cols, 3)
                    s_idx = jnp.bitwise_and(cols, 7)
                    for l0 in range(0, 128, 4 * _L):
                        lane_g = [
                            l0 + k * _L + lax.iota(jnp.int32, _L)
                            for k in range(4)
                        ]
                        w_g = [
                            plsc.load_gather(rows, [lanes, cols])
                            for lanes in lane_g
                        ]
                        for lanes, w32 in zip(lane_g, w_g):
                            plsc.store_scatter(
                                tr3,
                                [r_idx, s_idx, lanes],
                                lax.bitcast_convert_type(w32, jnp.float32),
                            )

            store_descr(g, tbuf).start()

        store_descr(T - 2, lax.rem(T - 2, 2)).wait()
        store_descr(T - 1, lax.rem(T - 1, 2)).wait()

    return k


def kernel(x, W):
    B, T = x.shape
    V, D = W.shape
    Wi = lax.bitcast_convert_type(W, jnp.int32)
    xT = x.T.astype(jnp.int32)
    out5 = _make_gather(B, T, D, V)(Wi, xT)
    return out5.transpose(2, 4, 0, 1, 3).reshape(B, T, D)


# speedup vs baseline: 11.5828x; 1.0094x over previous
import functools

import jax
import jax.numpy as jnp
from jax import lax
from jax.experimental import pallas as pl
from jax.experimental.pallas import tpu as pltpu
from jax.experimental.pallas import tpu_sc as plsc

_NC = 2
_NS = 16
_NW = _NC * _NS
_NBUF = 6
_L = 16


@functools.lru_cache(maxsize=None)
def _make_gather(B: int, T: int, D: int, V: int):
    assert B == 128 * _NW and D % 16 == 0
    DP = D
    R = D // 8
    mesh = plsc.VectorSubcoreMesh(core_axis_name="c", subcore_axis_name="s")

    @functools.partial(
        pl.kernel,
        out_type=jax.ShapeDtypeStruct((T, R, _NW, 8, 128), jnp.float32),
        mesh=mesh,
        scratch_types=[
            pltpu.VMEM((T, 128), jnp.int32),
            pltpu.VMEM((_NBUF, 128, DP), jnp.int32),
            pltpu.VMEM((2, R, 8, 128), jnp.float32),
            pltpu.SemaphoreType.DMA((_NBUF,)),
            pltpu.SemaphoreType.DMA((2,)),
        ],
        compiler_params=pltpu.CompilerParams(
            use_tc_tiling_on_sc=False, needs_layout_passes=False
        ),
    )
    def k(table_hbm, idx_hbm, out_hbm, idx_v, rows_v, tr_v, gsem, tsem):
        wid = lax.axis_index("s") * _NC + lax.axis_index("c")
        pltpu.sync_copy(idx_hbm.at[:, pl.ds(wid * 128, 128)], idx_v)

        def start_gather(g, buf):
            pltpu.async_copy(
                table_hbm.at[idx_v.at[g]], rows_v.at[buf], gsem.at[buf]
            )

        def store_descr(g, buf):
            return pltpu.make_async_copy(
                tr_v.at[buf], out_hbm.at[g, :, wid], tsem.at[buf]
            )

        for p in range(_NBUF - 1):
            start_gather(p, p)

        @pl.loop(0, T)
        def _(g):
            gbuf = lax.rem(g, _NBUF)
            tbuf = lax.rem(g, 2)

            @pl.when(g + _NBUF - 1 < T)
            def _():
                start_gather(g + _NBUF - 1, lax.rem(g + _NBUF - 1, _NBUF))

            pltpu.make_async_copy(
                table_hbm.at[idx_v.at[g]], rows_v.at[gbuf], gsem.at[gbuf]
            ).wait()

            @pl.when(g >= 2)
            def _():
                store_descr(g - 2, tbuf).wait()

            rows = rows_v.at[gbuf]
            tr3 = tr_v.at[tbuf]

            @pl.loop(0, _L)
            def _(j):
                iota = lax.iota(jnp.int32, _L)
                perm = jnp.bitwise_and(iota + j, _L - 1)
                for d0 in range(0, D, _L):
                    cols = d0 + perm
                    r_idx = lax.shift_right_logical(cols, 3)
                    s_idx = jnp.bitwise_and(cols, 7)
                    lane_g = [
                        l0 + lax.iota(jnp.int32, _L)
                        for l0 in range(0, 128, _L)
                    ]
                    w_g = [
                        plsc.load_gather(rows, [lanes, cols])
                        for lanes in lane_g
                    ]
                    for lanes, w32 in zip(lane_g, w_g):
                        plsc.store_scatter(
                            tr3,
                            [r_idx, s_idx, lanes],
                            lax.bitcast_convert_type(w32, jnp.float32),
                        )

            store_descr(g, tbuf).start()

        store_descr(T - 2, lax.rem(T - 2, 2)).wait()
        store_descr(T - 1, lax.rem(T - 1, 2)).wait()

    return k


def kernel(x, W):
    B, T = x.shape
    V, D = W.shape
    Wi = lax.bitcast_convert_type(W, jnp.int32)
    xT = x.T.astype(jnp.int32)
    out5 = _make_gather(B, T, D, V)(Wi, xT)
    return out5.transpose(2, 4, 0, 1, 3).reshape(B, T, D)
